# baseline (speedup 1.0000x reference)
import functools

import jax
import jax.numpy as jnp
from jax import lax
from jax.experimental import pallas as pl
from jax.experimental.pallas import tpu as pltpu
from jax.experimental.pallas import tpu_sc as plsc

NC = 2
NS = 16
NW = NC * NS
EB = 128


def _ceil_to(x, m):
    return (x + m - 1) // m * m


def _deg_sc(dst_r, ones_rows, zeros128, n_acc, blocks_per_tile, d):
    slab = n_acc // NS
    mesh = plsc.VectorSubcoreMesh(core_axis_name="c", subcore_axis_name="s")

    @functools.partial(
        pl.kernel,
        out_type=jax.ShapeDtypeStruct((NC, n_acc, d), jnp.float32),
        mesh=mesh,
        scratch_types=[
            pltpu.VMEM_SHARED((n_acc, d), jnp.float32),
            pltpu.VMEM((blocks_per_tile, EB), jnp.int32),
            pltpu.VMEM((EB, d), jnp.float32),
        ],
    )
    def k(dst_hbm, ones_hbm, zeros_hbm, parts_hbm, acc, idx_d, ones_v):
        c = lax.axis_index("c")
        s = lax.axis_index("s")
        wid = c * NS + s
        pltpu.sync_copy(dst_hbm.at[wid], idx_d)
        pltpu.sync_copy(ones_hbm, ones_v)
        pltpu.sync_copy(zeros_hbm, acc.at[pl.ds(s * slab, slab)])
        plsc.subcore_barrier()

        def step(j, carry):
            pltpu.sync_copy(ones_v, acc.at[idx_d.at[j]], add=True)
            return carry

        lax.fori_loop(0, blocks_per_tile, step, 0)
        plsc.subcore_barrier()
        pltpu.sync_copy(acc.at[pl.ds(s * slab, slab)],
                        parts_hbm.at[c, pl.ds(s * slab, slab)])

    return k(dst_r, ones_rows, zeros128)


NBUF = 2
SB = 8
ISLOT = 4

NJ_CORE0 = 20
NJ_CORE1 = 0


def _agg_sc(u, sd_r, zeros128, n_acc, d):
    slab = n_acc // NS
    nb_max = max(NJ_CORE0, NJ_CORE1) * SB
    mesh = plsc.VectorSubcoreMesh(core_axis_name="c", subcore_axis_name="s")

    @functools.partial(
        pl.kernel,
        out_type=jax.ShapeDtypeStruct((NC, n_acc, d), jnp.float32),
        mesh=mesh,
        scratch_types=[
            pltpu.VMEM_SHARED((n_acc, d), jnp.float32),
            pltpu.VMEM((ISLOT, SB, 2, EB), jnp.int32),
            [pltpu.VMEM((EB, d), jnp.float32) for _ in range(NBUF)],
            [pltpu.SemaphoreType.DMA for _ in range(ISLOT)],
            [pltpu.SemaphoreType.DMA for _ in range(NBUF)],
        ],
    )
    def k(u_hbm, sd_hbm, zeros_hbm, parts_hbm, acc, idx, rows, sems_i, sems_g):
        c = lax.axis_index("c")
        s = lax.axis_index("s")
        wid = c * NS + s
        pltpu.sync_copy(zeros_hbm, acc.at[pl.ds(s * slab, slab)])

        def sdma(jj, q):
            return pltpu.make_async_copy(
                sd_hbm.at[wid, pl.ds(jj * SB, SB)], idx.at[q], sems_i[q])

        def gather(b, q, r):
            return pltpu.make_async_copy(u_hbm.at[idx.at[q, r, 0]], rows[b],
                                         sems_g[b])

        plsc.subcore_barrier()

        def super_block(jbase, J, q, refill, wait_next):
            if refill:
                sdma(J + 2, (q + 2) % ISLOT).start()
            for r in range(SB):
                b = r % NBUF
                gather(b, q, r).wait()
                pltpu.sync_copy(rows[b], acc.at[idx.at[q, r, 1]], add=True)
                if r < SB - NBUF:
                    gather(b, q, r + NBUF).start()
                else:
                    if wait_next and r == SB - NBUF:
                        sdma(J + 1, (q + 1) % ISLOT).wait()
                    if wait_next:
                        gather(b, (q + 1) % ISLOT, r + NBUF - SB).start()

        def run(nj):
            if nj == 0:
                return
            sdma(0, 0).start()
            sdma(1, 1).start()
            sdma(0, 0).wait()
            gather(0, 0, 0).start()
            gather(1, 0, 1).start()
            rem = (nj - 2) % ISLOT
            steady = (nj - 2 - rem) // ISLOT

            def step(g, carry):
                for v in range(ISLOT):
                    J = g * ISLOT + v
                    super_block(J * SB, J, v, True, True)
                return carry

            if steady > 0:
                lax.fori_loop(0, steady, step, 0, unroll=False)
            for J in range(steady * ISLOT, nj):
                super_block(J * SB, J, J % ISLOT, J + 2 < nj, J < nj - 1)

        lax.cond(c == 0, lambda: run(NJ_CORE0), lambda: run(NJ_CORE1))
        plsc.subcore_barrier()
        pltpu.sync_copy(acc.at[pl.ds(s * slab, slab)],
                        parts_hbm.at[c, pl.ds(s * slab, slab)])

    return k(u, sd_r, zeros128)


def _prep_tc(feat, d0, d1, rows_blk):
    n, d = feat.shape
    grid = n // rows_blk

    def body(f_ref, d0_ref, d1_ref, u_ref, dv_ref):
        deg = 1.0 + d0_ref[:, :1] + d1_ref[:, :1]
        dinv = lax.rsqrt(deg)
        u_ref[...] = f_ref[...] * dinv
        dv_ref[...] = jnp.broadcast_to(dinv, f_ref.shape)

    return pl.pallas_call(
        body,
        grid=(grid,),
        in_specs=[
            pl.BlockSpec((rows_blk, d), lambda i: (i, 0)),
            pl.BlockSpec((rows_blk, d), lambda i: (i, 0)),
            pl.BlockSpec((rows_blk, d), lambda i: (i, 0)),
        ],
        out_specs=[
            pl.BlockSpec((rows_blk, d), lambda i: (i, 0)),
            pl.BlockSpec((rows_blk, d), lambda i: (i, 0)),
        ],
        out_shape=[
            jax.ShapeDtypeStruct((n, d), jnp.float32),
            jax.ShapeDtypeStruct((n, d), jnp.float32),
        ],
    )(feat, d0, d1)


def _combine_tc(p0, p1, u, dv, rows_blk):
    n, d = u.shape
    grid = n // rows_blk

    def body(p0_ref, p1_ref, u_ref, dv_ref, x_ref, un_ref):
        t = p0_ref[...] + p1_ref[...] + u_ref[...]
        x = jnp.maximum(dv_ref[...] * t, 0.0)
        x_ref[...] = x
        un_ref[...] = dv_ref[...] * x

    return pl.pallas_call(
        body,
        grid=(grid,),
        in_specs=[pl.BlockSpec((rows_blk, d), lambda i: (i, 0))] * 4,
        out_specs=[pl.BlockSpec((rows_blk, d), lambda i: (i, 0))] * 2,
        out_shape=[
            jax.ShapeDtypeStruct((n, d), jnp.float32),
            jax.ShapeDtypeStruct((n, d), jnp.float32),
        ],
    )(p0, p1, u, dv)


def kernel(features, edge, W1, W2, W3, b1, b2, b3):
    n, d = features.shape
    e = edge.shape[1]

    n_acc = _ceil_to(n + 1, NS * 8)

    e_pad = (NJ_CORE0 + NJ_CORE1) * NS * SB * EB
    assert e_pad >= e
    pad = e_pad - e
    src = edge[0].astype(jnp.int32)
    dst = edge[1].astype(jnp.int32)
    src_p = jnp.concatenate([src, jnp.zeros((pad,), jnp.int32)])
    dst_p = jnp.concatenate([dst, jnp.full((pad,), n_acc - 1, jnp.int32)])

    nb_max = max(NJ_CORE0, NJ_CORE1) * SB
    e0 = NJ_CORE0 * NS * SB * EB
    halves = []
    for nj, sl in ((NJ_CORE0, slice(0, e0)), (NJ_CORE1, slice(e0, e_pad))):
        s_c = src_p[sl].reshape(NS, nj * SB, EB)
        d_c = dst_p[sl].reshape(NS, nj * SB, EB)
        sd_c = jnp.stack([s_c, d_c], axis=2)
        sd_c = jnp.pad(sd_c, ((0, 0), (0, nb_max - nj * SB), (0, 0), (0, 0)))
        halves.append(sd_c)
    sd_r = jnp.concatenate(halves, axis=0)

    nb_deg = e_pad // (NW * EB)
    dst_r = dst_p.reshape(NW, nb_deg, EB)

    slab = n_acc // NS
    ones_rows = jnp.ones((EB, d), jnp.float32)
    zeros128 = jnp.zeros((slab, d), jnp.float32)

    deg_parts = _deg_sc(dst_r, ones_rows, zeros128, n_acc, nb_deg, d)
    u, dv = _prep_tc(features, deg_parts[0, :n], deg_parts[1, :n], 400)

    outs = [features]
    x = None
    for _ in range(3):
        parts = _agg_sc(u, sd_r, zeros128, n_acc, d)
        x, u = _combine_tc(parts[0, :n], parts[1, :n], u, dv, 400)
        outs.append(x)
    return jnp.concatenate(outs, axis=1)


# speedup vs baseline: 1.1219x; 1.1219x over previous
import functools

import jax
import jax.numpy as jnp
from jax import lax
from jax.experimental import pallas as pl
from jax.experimental.pallas import tpu as pltpu
from jax.experimental.pallas import tpu_sc as plsc

NC = 2
NS = 16
NW = NC * NS
EB = 128


def _ceil_to(x, m):
    return (x + m - 1) // m * m


def _deg_sc(dst_r, ones_rows, zeros128, n_acc, blocks_per_tile, d):
    slab = n_acc // NS
    mesh = plsc.VectorSubcoreMesh(core_axis_name="c", subcore_axis_name="s")

    @functools.partial(
        pl.kernel,
        out_type=jax.ShapeDtypeStruct((NC, n_acc, d), jnp.float32),
        mesh=mesh,
        scratch_types=[
            pltpu.VMEM_SHARED((n_acc, d), jnp.float32),
            pltpu.VMEM((blocks_per_tile, EB), jnp.int32),
            pltpu.VMEM((EB, d), jnp.float32),
        ],
    )
    def k(dst_hbm, ones_hbm, zeros_hbm, parts_hbm, acc, idx_d, ones_v):
        c = lax.axis_index("c")
        s = lax.axis_index("s")
        wid = c * NS + s
        pltpu.sync_copy(dst_hbm.at[wid], idx_d)
        pltpu.sync_copy(ones_hbm, ones_v)
        pltpu.sync_copy(zeros_hbm, acc.at[pl.ds(s * slab, slab)])
        plsc.subcore_barrier()

        def step(j, carry):
            pltpu.sync_copy(ones_v, acc.at[idx_d.at[j]], add=True)
            return carry

        lax.fori_loop(0, blocks_per_tile, step, 0)
        plsc.subcore_barrier()
        pltpu.sync_copy(acc.at[pl.ds(s * slab, slab)],
                        parts_hbm.at[c, pl.ds(s * slab, slab)])

    return k(dst_r, ones_rows, zeros128)


NBUF = 2
SB = 8
ISLOT = 4

NJ_CORE0 = 10
NJ_CORE1 = 10


def _agg_sc(u, sd_r, zeros128, n_acc, d):
    slab = n_acc // NS
    nb_max = max(NJ_CORE0, NJ_CORE1) * SB
    mesh = plsc.VectorSubcoreMesh(core_axis_name="c", subcore_axis_name="s")

    @functools.partial(
        pl.kernel,
        out_type=jax.ShapeDtypeStruct((NC, n_acc, d), jnp.float32),
        mesh=mesh,
        scratch_types=[
            pltpu.VMEM_SHARED((n_acc, d), jnp.float32),
            pltpu.VMEM((ISLOT, SB, 2, EB), jnp.int32),
            [pltpu.VMEM((EB, d), jnp.float32) for _ in range(NBUF)],
            [pltpu.SemaphoreType.DMA for _ in range(ISLOT)],
            [pltpu.SemaphoreType.DMA for _ in range(NBUF)],
        ],
    )
    def k(u_hbm, sd_hbm, zeros_hbm, parts_hbm, acc, idx, rows, sems_i, sems_g):
        c = lax.axis_index("c")
        s = lax.axis_index("s")
        wid = c * NS + s
        pltpu.sync_copy(zeros_hbm, acc.at[pl.ds(s * slab, slab)])

        def sdma(jj, q):
            return pltpu.make_async_copy(
                sd_hbm.at[wid, pl.ds(jj * SB, SB)], idx.at[q], sems_i[q])

        def gather(b, q, r):
            return pltpu.make_async_copy(u_hbm.at[idx.at[q, r, 0]], rows[b],
                                         sems_g[b])

        plsc.subcore_barrier()

        def super_block(jbase, J, q, refill, wait_next):
            if refill:
                sdma(J + 2, (q + 2) % ISLOT).start()
            for r in range(SB):
                b = r % NBUF
                gather(b, q, r).wait()
                pltpu.sync_copy(rows[b], acc.at[idx.at[q, r, 1]], add=True)
                if r < SB - NBUF:
                    gather(b, q, r + NBUF).start()
                else:
                    if wait_next and r == SB - NBUF:
                        sdma(J + 1, (q + 1) % ISLOT).wait()
                    if wait_next:
                        gather(b, (q + 1) % ISLOT, r + NBUF - SB).start()

        def run(nj):
            if nj == 0:
                return
            sdma(0, 0).start()
            sdma(1, 1).start()
            sdma(0, 0).wait()
            gather(0, 0, 0).start()
            gather(1, 0, 1).start()
            rem = (nj - 2) % ISLOT
            steady = (nj - 2 - rem) // ISLOT

            def step(g, carry):
                for v in range(ISLOT):
                    J = g * ISLOT + v
                    super_block(J * SB, J, v, True, True)
                return carry

            if steady > 0:
                lax.fori_loop(0, steady, step, 0, unroll=False)
            for J in range(steady * ISLOT, nj):
                super_block(J * SB, J, J % ISLOT, J + 2 < nj, J < nj - 1)

        lax.cond(c == 0, lambda: run(NJ_CORE0), lambda: run(NJ_CORE1))
        plsc.subcore_barrier()
        pltpu.sync_copy(acc.at[pl.ds(s * slab, slab)],
                        parts_hbm.at[c, pl.ds(s * slab, slab)])

    return k(u, sd_r, zeros128)


def _prep_tc(feat, d0, d1, rows_blk):
    n, d = feat.shape
    grid = n // rows_blk

    def body(f_ref, d0_ref, d1_ref, u_ref, dv_ref):
        deg = 1.0 + d0_ref[:, :1] + d1_ref[:, :1]
        dinv = lax.rsqrt(deg)
        u_ref[...] = f_ref[...] * dinv
        dv_ref[...] = jnp.broadcast_to(dinv, f_ref.shape)

    return pl.pallas_call(
        body,
        grid=(grid,),
        in_specs=[
            pl.BlockSpec((rows_blk, d), lambda i: (i, 0)),
            pl.BlockSpec((rows_blk, d), lambda i: (i, 0)),
            pl.BlockSpec((rows_blk, d), lambda i: (i, 0)),
        ],
        out_specs=[
            pl.BlockSpec((rows_blk, d), lambda i: (i, 0)),
            pl.BlockSpec((rows_blk, d), lambda i: (i, 0)),
        ],
        out_shape=[
            jax.ShapeDtypeStruct((n, d), jnp.float32),
            jax.ShapeDtypeStruct((n, d), jnp.float32),
        ],
    )(feat, d0, d1)


def _combine_tc(p0, p1, u, dv, rows_blk):
    n, d = u.shape
    grid = n // rows_blk

    def body(p0_ref, p1_ref, u_ref, dv_ref, x_ref, un_ref):
        t = p0_ref[...] + p1_ref[...] + u_ref[...]
        x = jnp.maximum(dv_ref[...] * t, 0.0)
        x_ref[...] = x
        un_ref[...] = dv_ref[...] * x

    return pl.pallas_call(
        body,
        grid=(grid,),
        in_specs=[pl.BlockSpec((rows_blk, d), lambda i: (i, 0))] * 4,
        out_specs=[pl.BlockSpec((rows_blk, d), lambda i: (i, 0))] * 2,
        out_shape=[
            jax.ShapeDtypeStruct((n, d), jnp.float32),
            jax.ShapeDtypeStruct((n, d), jnp.float32),
        ],
    )(p0, p1, u, dv)


def kernel(features, edge, W1, W2, W3, b1, b2, b3):
    n, d = features.shape
    e = edge.shape[1]

    n_acc = _ceil_to(n + 1, NS * 8)

    e_pad = (NJ_CORE0 + NJ_CORE1) * NS * SB * EB
    assert e_pad >= e
    pad = e_pad - e
    src = edge[0].astype(jnp.int32)
    dst = edge[1].astype(jnp.int32)
    src_p = jnp.concatenate([src, jnp.zeros((pad,), jnp.int32)])
    dst_p = jnp.concatenate([dst, jnp.full((pad,), n_acc - 1, jnp.int32)])

    nb_max = max(NJ_CORE0, NJ_CORE1) * SB
    e0 = NJ_CORE0 * NS * SB * EB
    halves = []
    for nj, sl in ((NJ_CORE0, slice(0, e0)), (NJ_CORE1, slice(e0, e_pad))):
        s_c = src_p[sl].reshape(NS, nj * SB, EB)
        d_c = dst_p[sl].reshape(NS, nj * SB, EB)
        sd_c = jnp.stack([s_c, d_c], axis=2)
        sd_c = jnp.pad(sd_c, ((0, 0), (0, nb_max - nj * SB), (0, 0), (0, 0)))
        halves.append(sd_c)
    sd_r = jnp.concatenate(halves, axis=0)

    nb_deg = e_pad // (NW * EB)
    dst_r = dst_p.reshape(NW, nb_deg, EB)

    slab = n_acc // NS
    ones_rows = jnp.ones((EB, d), jnp.float32)
    zeros128 = jnp.zeros((slab, d), jnp.float32)

    deg_parts = _deg_sc(dst_r, ones_rows, zeros128, n_acc, nb_deg, d)
    u, dv = _prep_tc(features, deg_parts[0, :n], deg_parts[1, :n], 400)

    outs = [features]
    x = None
    for _ in range(3):
        parts = _agg_sc(u, sd_r, zeros128, n_acc, d)
        x, u = _combine_tc(parts[0, :n], parts[1, :n], u, dv, 400)
        outs.append(x)
    return jnp.concatenate(outs, axis=1)


# speedup vs baseline: 1.3803x; 1.2303x over previous
import functools

import jax
import jax.numpy as jnp
from jax import lax
from jax.experimental import pallas as pl
from jax.experimental.pallas import tpu as pltpu
from jax.experimental.pallas import tpu_sc as plsc

NC = 2
NS = 16
NW = NC * NS
EB = 128


def _ceil_to(x, m):
    return (x + m - 1) // m * m


def _deg_sc(dst_r, ones_rows, zeros128, n_acc, blocks_per_tile, d):
    slab = n_acc // NS
    mesh = plsc.VectorSubcoreMesh(core_axis_name="c", subcore_axis_name="s")

    @functools.partial(
        pl.kernel,
        out_type=jax.ShapeDtypeStruct((NC, n_acc, d), jnp.float32),
        mesh=mesh,
        scratch_types=[
            pltpu.VMEM_SHARED((n_acc, d), jnp.float32),
            pltpu.VMEM((blocks_per_tile, EB), jnp.int32),
            pltpu.VMEM((EB, d), jnp.float32),
        ],
    )
    def k(dst_hbm, ones_hbm, zeros_hbm, parts_hbm, acc, idx_d, ones_v):
        c = lax.axis_index("c")
        s = lax.axis_index("s")
        wid = c * NS + s
        pltpu.sync_copy(dst_hbm.at[wid], idx_d)
        pltpu.sync_copy(ones_hbm, ones_v)
        pltpu.sync_copy(zeros_hbm, acc.at[pl.ds(s * slab, slab)])
        plsc.subcore_barrier()

        def step(j, carry):
            pltpu.sync_copy(ones_v, acc.at[idx_d.at[j]], add=True)
            return carry

        lax.fori_loop(0, blocks_per_tile, step, 0)
        plsc.subcore_barrier()
        pltpu.sync_copy(acc.at[pl.ds(s * slab, slab)],
                        parts_hbm.at[c, pl.ds(s * slab, slab)])

    return k(dst_r, ones_rows, zeros128)


NBUF = 2
SB = 8
ISLOT = 4


def _agg_sc(u, src_r, dst_r, zeros128, n_acc, blocks_per_tile, d):
    slab = n_acc // NS
    nb = blocks_per_tile
    nj = nb // SB
    assert nb % SB == 0 and nj % ISLOT == 2 and nj >= 6
    mesh = plsc.VectorSubcoreMesh(core_axis_name="c", subcore_axis_name="s")

    @functools.partial(
        pl.kernel,
        out_type=jax.ShapeDtypeStruct((NC, n_acc, d), jnp.float32),
        mesh=mesh,
        scratch_types=[
            pltpu.VMEM_SHARED((n_acc, d), jnp.float32),
            pltpu.VMEM((nb, EB), jnp.int32),
            pltpu.VMEM((ISLOT, SB, EB), jnp.int32),
            [pltpu.VMEM((EB, d), jnp.float32) for _ in range(NBUF)],
            [pltpu.SemaphoreType.DMA for _ in range(ISLOT)],
            [pltpu.SemaphoreType.DMA for _ in range(NBUF)],
        ],
    )
    def k(u_hbm, src_hbm, dst_hbm, zeros_hbm, parts_hbm,
          acc, idx_d, idx_s, rows, sems_i, sems_g):
        c = lax.axis_index("c")
        s = lax.axis_index("s")
        wid = c * NS + s
        pltpu.sync_copy(dst_hbm.at[wid], idx_d)
        pltpu.sync_copy(zeros_hbm, acc.at[pl.ds(s * slab, slab)])

        def sdma(jj, q):
            return pltpu.make_async_copy(src_hbm.at[wid, pl.ds(jj * SB, SB)],
                                         idx_s.at[q], sems_i[q])

        def gather(b, q, r):
            return pltpu.make_async_copy(u_hbm.at[idx_s.at[q, r]], rows[b],
                                         sems_g[b])

        sdma(0, 0).start()
        sdma(1, 1).start()
        sdma(0, 0).wait()
        gather(0, 0, 0).start()
        gather(1, 0, 1).start()
        plsc.subcore_barrier()

        def super_block(jbase, q, refill, wait_next):
            if refill:
                sdma_next = sdma(jbase // SB + 2, (q + 2) % ISLOT)
                sdma_next.start()
            for r in range(SB):
                j = jbase + r
                b = r % NBUF
                gather(b, q, r).wait()
                pltpu.sync_copy(rows[b], acc.at[idx_d.at[j]], add=True)
                if r < SB - NBUF:
                    gather(b, q, r + NBUF).start()
                else:
                    if wait_next and r == SB - NBUF:
                        sdma(jbase // SB + 1, (q + 1) % ISLOT).wait()
                    if wait_next:
                        gather(b, (q + 1) % ISLOT, r + NBUF - SB).start()

        def step(g, carry):
            for v in range(ISLOT):
                super_block((g * ISLOT + v) * SB, v, True, True)
            return carry

        lax.fori_loop(0, (nj - 2) // ISLOT, step, 0, unroll=False)
        super_block((nj - 2) * SB, (nj - 2) % ISLOT, False, True)
        super_block((nj - 1) * SB, (nj - 1) % ISLOT, False, False)
        plsc.subcore_barrier()
        pltpu.sync_copy(acc.at[pl.ds(s * slab, slab)],
                        parts_hbm.at[c, pl.ds(s * slab, slab)])

    return k(u, src_r, dst_r, zeros128)


def _prep_tc(feat, d0, d1, rows_blk):
    n, d = feat.shape
    grid = n // rows_blk

    def body(f_ref, d0_ref, d1_ref, u_ref, dv_ref):
        deg = 1.0 + d0_ref[:, :1] + d1_ref[:, :1]
        dinv = lax.rsqrt(deg)
        u_ref[...] = f_ref[...] * dinv
        dv_ref[...] = jnp.broadcast_to(dinv, f_ref.shape)

    return pl.pallas_call(
        body,
        grid=(grid,),
        in_specs=[
            pl.BlockSpec((rows_blk, d), lambda i: (i, 0)),
            pl.BlockSpec((rows_blk, d), lambda i: (i, 0)),
            pl.BlockSpec((rows_blk, d), lambda i: (i, 0)),
        ],
        out_specs=[
            pl.BlockSpec((rows_blk, d), lambda i: (i, 0)),
            pl.BlockSpec((rows_blk, d), lambda i: (i, 0)),
        ],
        out_shape=[
            jax.ShapeDtypeStruct((n, d), jnp.float32),
            jax.ShapeDtypeStruct((n, d), jnp.float32),
        ],
    )(feat, d0, d1)


def _combine_tc(p0, p1, u, dv, rows_blk):
    n, d = u.shape
    grid = n // rows_blk

    def body(p0_ref, p1_ref, u_ref, dv_ref, x_ref, un_ref):
        t = p0_ref[...] + p1_ref[...] + u_ref[...]
        x = jnp.maximum(dv_ref[...] * t, 0.0)
        x_ref[...] = x
        un_ref[...] = dv_ref[...] * x

    return pl.pallas_call(
        body,
        grid=(grid,),
        in_specs=[pl.BlockSpec((rows_blk, d), lambda i: (i, 0))] * 4,
        out_specs=[pl.BlockSpec((rows_blk, d), lambda i: (i, 0))] * 2,
        out_shape=[
            jax.ShapeDtypeStruct((n, d), jnp.float32),
            jax.ShapeDtypeStruct((n, d), jnp.float32),
        ],
    )(p0, p1, u, dv)


def kernel(features, edge, W1, W2, W3, b1, b2, b3):
    n, d = features.shape
    e = edge.shape[1]

    n_acc = _ceil_to(n + 1, NS * 8)
    e_pad = _ceil_to(e, NW * EB * SB)
    while (e_pad // (NW * EB * SB)) % ISLOT != 2:
        e_pad += NW * EB * SB
    blocks_per_tile = e_pad // (NW * EB)
    pad = e_pad - e

    src = edge[0].astype(jnp.int32)
    dst = edge[1].astype(jnp.int32)
    src_p = jnp.concatenate([src, jnp.zeros((pad,), jnp.int32)])
    dst_p = jnp.concatenate([dst, jnp.full((pad,), n_acc - 1, jnp.int32)])
    src_r = src_p.reshape(NW, blocks_per_tile, EB)
    dst_r = dst_p.reshape(NW, blocks_per_tile, EB)

    slab = n_acc // NS
    ones_rows = jnp.ones((EB, d), jnp.float32)
    zeros128 = jnp.zeros((slab, d), jnp.float32)

    deg_parts = _deg_sc(dst_r, ones_rows, zeros128, n_acc, blocks_per_tile, d)
    u, dv = _prep_tc(features, deg_parts[0, :n], deg_parts[1, :n], 400)

    outs = [features]
    x = None
    for _ in range(3):
        parts = _agg_sc(u, src_r, dst_r, zeros128, n_acc, blocks_per_tile, d)
        x, u = _combine_tc(parts[0, :n], parts[1, :n], u, dv, 400)
        outs.append(x)
    return jnp.concatenate(outs, axis=1)
